# baseline (device time: 118429 ns/iter reference)
import jax
import jax.numpy as jnp
from jax import lax
from jax.experimental import pallas as pl
from jax.experimental.pallas import tpu as pltpu

M = 2048
D = 2048
HALF = M // 2
NCHUNK = 16
CHUNK = HALF // NCHUNK
NSLOT = 4


def kernel(partial, resid, gamma):
    p = partial.reshape(M, D)
    g = gamma.reshape(1, D)

    def body(
        p_ref, r_ref, g_ref, out_ref,
        xbuf, ybuf, sendbuf, pv_m, pv_o, rv_m, rv_o, sb_m, sb_o,
        send_x, recv_x, send_y, recv_y,
        ls, lp_m, lp_o, lr_m, lr_o, st_m, st_o,
    ):
        my_x = lax.axis_index("x")
        my_y = lax.axis_index("y")
        my_z = lax.axis_index("z")
        xp = (1 - my_x, my_y, my_z)
        yp = (my_x, 1 - my_y, my_z)

        mine0 = my_y * HALF
        other0 = (1 - my_y) * HALF

        def rows_mine(k):
            return pl.ds(mine0 + k * CHUNK, CHUNK)

        def rows_other(k):
            return pl.ds(other0 + k * CHUNK, CHUNK)

        ls_d = []
        for k in range(NCHUNK):
            cp = pltpu.make_async_copy(
                p_ref.at[rows_mine(k), :], sendbuf.at[k], ls.at[k]
            )
            cp.start()
            ls_d.append(cp)

        barrier = pltpu.get_barrier_semaphore()
        for nbr in (xp, yp):
            pl.semaphore_signal(
                barrier, inc=1, device_id=nbr,
                device_id_type=pl.DeviceIdType.MESH,
            )
        pl.semaphore_wait(barrier, 2)

        x_sends = []
        for k in range(NCHUNK):
            ls_d[k].wait()
            rd = pltpu.make_async_remote_copy(
                src_ref=sendbuf.at[k],
                dst_ref=xbuf.at[k],
                send_sem=send_x.at[k],
                recv_sem=recv_x.at[k],
                device_id=xp,
                device_id_type=pl.DeviceIdType.MESH,
            )
            rd.start()
            x_sends.append(rd)

        def make_fwd(k):
            return pltpu.make_async_remote_copy(
                src_ref=xbuf.at[k],
                dst_ref=ybuf.at[k],
                send_sem=send_y.at[k],
                recv_sem=recv_y.at[k],
                device_id=yp,
                device_id_type=pl.DeviceIdType.MESH,
            )

        def load(dst, slot, src_rows, src, sem, k):
            cp = pltpu.make_async_copy(
                src.at[src_rows, :], dst.at[slot], sem.at[k]
            )
            cp.start()
            return cp

        lp_m_d = [None] * NCHUNK
        lp_o_d = [None] * NCHUNK
        lr_m_d = [None] * NCHUNK
        lr_o_d = [None] * NCHUNK
        st_m_d = [None] * NCHUNK
        st_o_d = [None] * NCHUNK

        def prefetch_mine(k):
            if k < NCHUNK:
                lp_m_d[k] = load(pv_m, k % NSLOT, rows_mine(k), p_ref, lp_m, k)
                lr_m_d[k] = load(rv_m, k % NSLOT, rows_mine(k), r_ref, lr_m, k)

        def prefetch_other(k):
            if k < NCHUNK:
                lp_o_d[k] = load(pv_o, k % NSLOT, rows_other(k), p_ref, lp_o, k)
                lr_o_d[k] = load(rv_o, k % NSLOT, rows_other(k), r_ref, lr_o, k)

        def ln(recv_chunk, pv, rv):
            yv = recv_chunk + pv + rv
            ms = jnp.sum(yv * yv, axis=1, keepdims=True) * (1.0 / D) + 1e-6
            return yv * lax.rsqrt(ms) * g_ref[0:1, :]

        def compute_mine(j):
            lp_m_d[j].wait()
            lr_m_d[j].wait()
            if j >= NSLOT:
                st_m_d[j - NSLOT].wait()
            s = j % NSLOT
            sb_m[s] = ln(xbuf[j], pv_m[s], rv_m[s])
            cp = pltpu.make_async_copy(
                sb_m.at[s], out_ref.at[rows_mine(j), :], st_m.at[j]
            )
            cp.start()
            st_m_d[j] = cp

        def compute_other(j):
            pltpu.make_async_remote_copy(
                src_ref=xbuf.at[j],
                dst_ref=ybuf.at[j],
                send_sem=send_y.at[j],
                recv_sem=recv_y.at[j],
                device_id=yp,
                device_id_type=pl.DeviceIdType.MESH,
            ).wait_recv()
            lp_o_d[j].wait()
            lr_o_d[j].wait()
            if j >= NSLOT:
                st_o_d[j - NSLOT].wait()
            s = j % NSLOT
            sb_o[s] = ln(ybuf[j], pv_o[s], rv_o[s])
            cp = pltpu.make_async_copy(
                sb_o.at[s], out_ref.at[rows_other(j), :], st_o.at[j]
            )
            cp.start()
            st_o_d[j] = cp

        prefetch_mine(0)
        prefetch_mine(1)
        prefetch_other(0)

        y_fwds = [None] * NCHUNK
        for k in range(NCHUNK):
            prefetch_mine(k + 2)
            prefetch_other(k + 1)
            x_sends[k].wait_recv()
            y_fwds[k] = make_fwd(k)
            y_fwds[k].start()
            if k >= 1:
                compute_mine(k - 1)
            if k >= 2:
                compute_other(k - 2)

        compute_mine(NCHUNK - 1)
        compute_other(NCHUNK - 2)
        compute_other(NCHUNK - 1)
        for k in range(NCHUNK):
            x_sends[k].wait_send()
            y_fwds[k].wait_send()
        for j in range(NCHUNK - NSLOT, NCHUNK):
            st_m_d[j].wait()
            st_o_d[j].wait()

    return pl.pallas_call(
        body,
        out_shape=jax.ShapeDtypeStruct((M, D), jnp.float32),
        in_specs=[
            pl.BlockSpec(memory_space=pl.ANY),
            pl.BlockSpec(memory_space=pl.ANY),
            pl.BlockSpec(memory_space=pltpu.VMEM),
        ],
        out_specs=pl.BlockSpec(memory_space=pl.ANY),
        scratch_shapes=[
            pltpu.VMEM((NCHUNK, CHUNK, D), jnp.float32),
            pltpu.VMEM((NCHUNK, CHUNK, D), jnp.float32),
            pltpu.VMEM((NCHUNK, CHUNK, D), jnp.float32),
            pltpu.VMEM((NSLOT, CHUNK, D), jnp.float32),
            pltpu.VMEM((NSLOT, CHUNK, D), jnp.float32),
            pltpu.VMEM((NSLOT, CHUNK, D), jnp.float32),
            pltpu.VMEM((NSLOT, CHUNK, D), jnp.float32),
            pltpu.VMEM((NSLOT, CHUNK, D), jnp.float32),
            pltpu.VMEM((NSLOT, CHUNK, D), jnp.float32),
            pltpu.SemaphoreType.DMA((NCHUNK,)),
            pltpu.SemaphoreType.DMA((NCHUNK,)),
            pltpu.SemaphoreType.DMA((NCHUNK,)),
            pltpu.SemaphoreType.DMA((NCHUNK,)),
            pltpu.SemaphoreType.DMA((NCHUNK,)),
            pltpu.SemaphoreType.DMA((NCHUNK,)),
            pltpu.SemaphoreType.DMA((NCHUNK,)),
            pltpu.SemaphoreType.DMA((NCHUNK,)),
            pltpu.SemaphoreType.DMA((NCHUNK,)),
            pltpu.SemaphoreType.DMA((NCHUNK,)),
            pltpu.SemaphoreType.DMA((NCHUNK,)),
        ],
        compiler_params=pltpu.CompilerParams(
            collective_id=0, vmem_limit_bytes=100 * 1024 * 1024
        ),
    )(p, resid, g)


# device time: 116714 ns/iter; 1.0147x vs baseline; 1.0147x over previous
import jax
import jax.numpy as jnp
from jax import lax
from jax.experimental import pallas as pl
from jax.experimental.pallas import tpu as pltpu

M = 2048
D = 2048
HALF = M // 2
NCHUNK = 32
CHUNK = HALF // NCHUNK
NSLOT = 4


def kernel(partial, resid, gamma):
    p = partial.reshape(M, D)
    g = gamma.reshape(1, D)

    def body(
        p_ref, r_ref, g_ref, out_ref,
        xbuf, ybuf, sendbuf, pv_m, pv_o, rv_m, rv_o, sb_m, sb_o,
        send_x, recv_x, send_y, recv_y,
        ls, lp_m, lp_o, lr_m, lr_o, st_m, st_o,
    ):
        my_x = lax.axis_index("x")
        my_y = lax.axis_index("y")
        my_z = lax.axis_index("z")
        xp = (1 - my_x, my_y, my_z)
        yp = (my_x, 1 - my_y, my_z)

        mine0 = my_y * HALF
        other0 = (1 - my_y) * HALF

        def rows_mine(k):
            return pl.ds(mine0 + k * CHUNK, CHUNK)

        def rows_other(k):
            return pl.ds(other0 + k * CHUNK, CHUNK)

        ls_d = []
        for k in range(NCHUNK):
            cp = pltpu.make_async_copy(
                p_ref.at[rows_mine(k), :], sendbuf.at[k], ls.at[k]
            )
            cp.start()
            ls_d.append(cp)

        barrier = pltpu.get_barrier_semaphore()
        for nbr in (xp, yp):
            pl.semaphore_signal(
                barrier, inc=1, device_id=nbr,
                device_id_type=pl.DeviceIdType.MESH,
            )
        pl.semaphore_wait(barrier, 2)

        x_sends = []
        for k in range(NCHUNK):
            ls_d[k].wait()
            rd = pltpu.make_async_remote_copy(
                src_ref=sendbuf.at[k],
                dst_ref=xbuf.at[k],
                send_sem=send_x.at[k],
                recv_sem=recv_x.at[k],
                device_id=xp,
                device_id_type=pl.DeviceIdType.MESH,
            )
            rd.start()
            x_sends.append(rd)

        def make_fwd(k):
            return pltpu.make_async_remote_copy(
                src_ref=xbuf.at[k],
                dst_ref=ybuf.at[k],
                send_sem=send_y.at[k],
                recv_sem=recv_y.at[k],
                device_id=yp,
                device_id_type=pl.DeviceIdType.MESH,
            )

        def load(dst, slot, src_rows, src, sem, k):
            cp = pltpu.make_async_copy(
                src.at[src_rows, :], dst.at[slot], sem.at[k]
            )
            cp.start()
            return cp

        lp_m_d = [None] * NCHUNK
        lp_o_d = [None] * NCHUNK
        lr_m_d = [None] * NCHUNK
        lr_o_d = [None] * NCHUNK
        st_m_d = [None] * NCHUNK
        st_o_d = [None] * NCHUNK

        def prefetch_mine(k):
            if k < NCHUNK:
                lp_m_d[k] = load(pv_m, k % NSLOT, rows_mine(k), p_ref, lp_m, k)
                lr_m_d[k] = load(rv_m, k % NSLOT, rows_mine(k), r_ref, lr_m, k)

        def prefetch_other(k):
            if k < NCHUNK:
                lp_o_d[k] = load(pv_o, k % NSLOT, rows_other(k), p_ref, lp_o, k)
                lr_o_d[k] = load(rv_o, k % NSLOT, rows_other(k), r_ref, lr_o, k)

        def ln(recv_chunk, pv, rv):
            yv = recv_chunk + pv + rv
            ms = jnp.sum(yv * yv, axis=1, keepdims=True) * (1.0 / D) + 1e-6
            return yv * lax.rsqrt(ms) * g_ref[0:1, :]

        def compute_mine(j):
            lp_m_d[j].wait()
            lr_m_d[j].wait()
            if j >= NSLOT:
                st_m_d[j - NSLOT].wait()
            s = j % NSLOT
            sb_m[s] = ln(xbuf[j], pv_m[s], rv_m[s])
            cp = pltpu.make_async_copy(
                sb_m.at[s], out_ref.at[rows_mine(j), :], st_m.at[j]
            )
            cp.start()
            st_m_d[j] = cp

        def compute_other(j):
            pltpu.make_async_remote_copy(
                src_ref=xbuf.at[j],
                dst_ref=ybuf.at[j],
                send_sem=send_y.at[j],
                recv_sem=recv_y.at[j],
                device_id=yp,
                device_id_type=pl.DeviceIdType.MESH,
            ).wait_recv()
            lp_o_d[j].wait()
            lr_o_d[j].wait()
            if j >= NSLOT:
                st_o_d[j - NSLOT].wait()
            s = j % NSLOT
            sb_o[s] = ln(ybuf[j], pv_o[s], rv_o[s])
            cp = pltpu.make_async_copy(
                sb_o.at[s], out_ref.at[rows_other(j), :], st_o.at[j]
            )
            cp.start()
            st_o_d[j] = cp

        prefetch_mine(0)
        prefetch_mine(1)
        prefetch_other(0)

        y_fwds = [None] * NCHUNK
        for k in range(NCHUNK):
            prefetch_mine(k + 2)
            prefetch_other(k + 1)
            x_sends[k].wait_recv()
            y_fwds[k] = make_fwd(k)
            y_fwds[k].start()
            if k >= 1:
                compute_mine(k - 1)
            if k >= 2:
                compute_other(k - 2)

        compute_mine(NCHUNK - 1)
        compute_other(NCHUNK - 2)
        compute_other(NCHUNK - 1)
        for k in range(NCHUNK):
            x_sends[k].wait_send()
            y_fwds[k].wait_send()
        for j in range(NCHUNK - NSLOT, NCHUNK):
            st_m_d[j].wait()
            st_o_d[j].wait()

    return pl.pallas_call(
        body,
        out_shape=jax.ShapeDtypeStruct((M, D), jnp.float32),
        in_specs=[
            pl.BlockSpec(memory_space=pl.ANY),
            pl.BlockSpec(memory_space=pl.ANY),
            pl.BlockSpec(memory_space=pltpu.VMEM),
        ],
        out_specs=pl.BlockSpec(memory_space=pltpu.MemorySpace.HBM),
        scratch_shapes=[
            pltpu.VMEM((NCHUNK, CHUNK, D), jnp.float32),
            pltpu.VMEM((NCHUNK, CHUNK, D), jnp.float32),
            pltpu.VMEM((NCHUNK, CHUNK, D), jnp.float32),
            pltpu.VMEM((NSLOT, CHUNK, D), jnp.float32),
            pltpu.VMEM((NSLOT, CHUNK, D), jnp.float32),
            pltpu.VMEM((NSLOT, CHUNK, D), jnp.float32),
            pltpu.VMEM((NSLOT, CHUNK, D), jnp.float32),
            pltpu.VMEM((NSLOT, CHUNK, D), jnp.float32),
            pltpu.VMEM((NSLOT, CHUNK, D), jnp.float32),
            pltpu.SemaphoreType.DMA((NCHUNK,)),
            pltpu.SemaphoreType.DMA((NCHUNK,)),
            pltpu.SemaphoreType.DMA((NCHUNK,)),
            pltpu.SemaphoreType.DMA((NCHUNK,)),
            pltpu.SemaphoreType.DMA((NCHUNK,)),
            pltpu.SemaphoreType.DMA((NCHUNK,)),
            pltpu.SemaphoreType.DMA((NCHUNK,)),
            pltpu.SemaphoreType.DMA((NCHUNK,)),
            pltpu.SemaphoreType.DMA((NCHUNK,)),
            pltpu.SemaphoreType.DMA((NCHUNK,)),
            pltpu.SemaphoreType.DMA((NCHUNK,)),
        ],
        compiler_params=pltpu.CompilerParams(
            collective_id=0, vmem_limit_bytes=100 * 1024 * 1024
        ),
    )(p, resid, g)
